# baseline (speedup 1.0000x reference)
import functools

import jax
import jax.numpy as jnp
from jax import lax
from jax.experimental import pallas as pl
from jax.experimental.pallas import tpu as pltpu
from jax.experimental.pallas import tpu_sc as plsc

F32 = jnp.float32
HID = 16
CHUNK = 128
NW = 32


def _sc_gather(table, src1, eat=None):
    n_nodes, ncol = table.shape
    n_edges = src1.shape[0]
    slab = n_edges // NW
    sub = 1000 if ncol > 16 else slab
    nsub = slab // sub
    gchunk = 40
    with_ea = eat is not None
    mesh = plsc.VectorSubcoreMesh(core_axis_name="c", subcore_axis_name="s")

    out_type = jax.ShapeDtypeStruct((n_edges, ncol), F32)
    scratch = [
        pltpu.VMEM((slab,), jnp.int32),
        pltpu.VMEM((sub, ncol), F32),
        pltpu.SemaphoreType.DMA,
    ]
    if with_ea:
        out_type = [out_type, jax.ShapeDtypeStruct((n_edges * HID,), F32)]
        scratch.extend(pltpu.VMEM((slab + 128,), F32) for _ in range(4))
        scratch.append(pltpu.VMEM((sub * HID,), F32))

    @functools.partial(
        pl.kernel,
        out_type=out_type,
        mesh=mesh,
        compiler_params=pltpu.CompilerParams(use_tc_tiling_on_sc=False,
                                             needs_layout_passes=False),
        scratch_types=scratch,
    )
    def gather_kernel(table_hbm, idx_hbm, *refs):
        if with_ea:
            (eat_hbm, out_hbm, ea_hbm, idx_v, rows_v, sem,
             ab0, ab1, ab2, ab3, padbuf) = refs
            abufs = (ab0, ab1, ab2, ab3)
        else:
            out_hbm, idx_v, rows_v, sem = refs
        c = lax.axis_index("c")
        s = lax.axis_index("s")
        w = s * 2 + c
        base = w * slab
        pltpu.sync_copy(idx_hbm.at[pl.ds(base, slab)], idx_v)
        if with_ea:
            for d in range(4):
                pltpu.sync_copy(eat_hbm.at[d, pl.ds(base, slab)],
                                abufs[d].at[pl.ds(0, slab)])

        def sub_pass(q, carry):
            def fire(k, cr):
                o = q * sub + k * gchunk
                pltpu.async_copy(
                    table_hbm.at[idx_v.at[pl.ds(o, gchunk)]],
                    rows_v.at[pl.ds(k * gchunk, gchunk)], sem)
                return cr

            def drain(k, cr):
                o = q * sub + k * gchunk
                pltpu.make_async_copy(
                    table_hbm.at[idx_v.at[pl.ds(o, gchunk)]],
                    rows_v.at[pl.ds(k * gchunk, gchunk)], sem).wait()
                return cr

            lax.fori_loop(0, sub // gchunk, fire, 0, unroll=False)
            if with_ea:
                def eabody(jv, cr):
                    eidx = jv * 16 + lax.iota(jnp.int32, 16)
                    msk = eidx < sub
                    for d in range(4):
                        vec = abufs[d][pl.ds(q * sub + jv * 16, 16)]
                        plsc.store_scatter(padbuf, [eidx * HID + d], vec,
                                           mask=msk)
                    return cr

                lax.fori_loop(0, (sub + 15) // 16, eabody, 0, unroll=False)
            lax.fori_loop(0, sub // gchunk, drain, 0, unroll=False)
            pltpu.sync_copy(rows_v,
                            out_hbm.at[pl.ds(base + q * sub, sub)])
            if with_ea:
                pltpu.sync_copy(
                    padbuf,
                    ea_hbm.at[pl.ds((base + q * sub) * HID, sub * HID)])
            return carry

        if with_ea:
            def zfill(i, carry):
                padbuf[pl.ds(i * 16, 16)] = jnp.zeros((16,), F32)
                return carry

            lax.fori_loop(0, sub * HID // 16, zfill, 0, unroll=8)
        lax.fori_loop(0, nsub, sub_pass, 0, unroll=False)

    if with_ea:
        return gather_kernel(table, src1, eat)
    return gather_kernel(table, src1)


def _sc_scatter(msg, dst2, n_nodes, with_counts):
    nrows = dst2.shape[0]
    base = nrows // NW
    rem = nrows - base * NW
    slab = n_nodes // 16
    piece = 125
    mesh = plsc.VectorSubcoreMesh(core_axis_name="c", subcore_axis_name="s")

    out_type = [jax.ShapeDtypeStruct((2, n_nodes, HID), F32)]
    scratch = [
        pltpu.VMEM((CHUNK,), jnp.int32),
        pltpu.VMEM((CHUNK, HID), F32),
        pltpu.VMEM_SHARED((n_nodes, HID), F32),
    ]
    if with_counts:
        out_type.append(jax.ShapeDtypeStruct((2, n_nodes, HID), F32))
        scratch.append(pltpu.VMEM((CHUNK, HID), F32))
        scratch.append(pltpu.VMEM_SHARED((n_nodes, HID), F32))

    @functools.partial(pl.kernel, out_type=out_type, mesh=mesh,
                       compiler_params=pltpu.CompilerParams(
                           use_tc_tiling_on_sc=False),
                       scratch_types=scratch)
    def scatter_kernel(msg_hbm, dst_hbm, *refs):
        if with_counts:
            out_hbm, cnt_hbm, idx_v, row_v, acc, one_v, cacc = refs
        else:
            out_hbm, idx_v, row_v, acc = refs
        c = lax.axis_index("c")
        s = lax.axis_index("s")
        w = s * 2 + c

        def fill(i, carry):
            row_v[i] = jnp.zeros((HID,), F32)
            if with_counts:
                one_v[i] = jnp.full((HID,), 1.0, F32)
            return carry

        lax.fori_loop(0, CHUNK, fill, 0, unroll=False)
        for q in range(slab // piece):
            dst_slice = pl.ds(s * slab + q * piece, piece)
            pltpu.sync_copy(row_v.at[pl.ds(0, piece)], acc.at[dst_slice])
            if with_counts:
                pltpu.sync_copy(row_v.at[pl.ds(0, piece)], cacc.at[dst_slice])
        plsc.subcore_barrier()

        def body(j, carry):
            r = w + NW * j
            pltpu.sync_copy(dst_hbm.at[r], idx_v)
            pltpu.sync_copy(msg_hbm.at[pl.ds(r * CHUNK, CHUNK)], row_v)
            pltpu.sync_copy(row_v, acc.at[idx_v], add=True)
            if with_counts:
                pltpu.sync_copy(one_v, cacc.at[idx_v], add=True)
            return carry

        lax.fori_loop(0, base, body, 0, unroll=False)
        if rem:
            @pl.when(w < rem)
            def _():
                body(base, 0)
        plsc.subcore_barrier()
        pltpu.sync_copy(acc.at[pl.ds(s * slab, slab)],
                        out_hbm.at[c, pl.ds(s * slab, slab)])
        if with_counts:
            pltpu.sync_copy(cacc.at[pl.ds(s * slab, slab)],
                            cnt_hbm.at[c, pl.ds(s * slab, slab)])

    return scatter_kernel(msg, dst2)


MXT = jnp.float32


def _ssum_mat():
    kk = jnp.arange(HID * HID, dtype=jnp.int32)
    return (kk[:, None] % HID == jnp.arange(HID, dtype=jnp.int32)[None, :]
            ).astype(F32)


def _rep_mat():
    kk = jnp.arange(HID * HID, dtype=jnp.int32)
    return (kk[None, :] // HID == jnp.arange(HID, dtype=jnp.int32)[:, None]
            ).astype(F32)


def _tc_he(ea8, ew1, eb1):
    rows = ea8.shape[0]
    tile = 2000
    ew1p = jnp.zeros((HID, HID), F32).at[:4].set(ew1)
    w = jnp.kron(jnp.eye(8, dtype=F32), ew1p).astype(MXT)
    b = jnp.tile(eb1, 8).reshape(1, 128)

    def body(ea_ref, w_ref, b_ref, out_ref):
        out_ref[...] = jnp.maximum(
            jnp.dot(ea_ref[...].astype(MXT), w_ref[...],
                    preferred_element_type=F32) + b_ref[...], 0.0)

    return pl.pallas_call(
        body,
        grid=(rows // tile,),
        in_specs=[
            pl.BlockSpec((tile, 128), lambda i: (i, 0)),
            pl.BlockSpec((128, 128), lambda i: (0, 0)),
            pl.BlockSpec((1, 128), lambda i: (0, 0)),
        ],
        out_specs=pl.BlockSpec((tile, 128), lambda i: (i, 0)),
        out_shape=jax.ShapeDtypeStruct((rows, 128), F32),
    )(ea8, w, b)


def _tc_edge(hg, att, attw, attb, att_relu, w2m, b2m, pack, tile_rows):
    n_edges, in_c = hg.shape
    rows = n_edges // pack
    grid = rows // tile_rows
    eye = jnp.eye(pack, dtype=F32)
    w2big = jnp.kron(eye, w2m).astype(MXT)
    b2big = jnp.kron(eye, b2m).astype(MXT)
    sbig = jnp.kron(eye, _ssum_mat()).astype(MXT)
    hg_p = hg.reshape(rows, pack * in_c)
    attk = attw.shape[0]
    wk = pack * HID * HID

    def body(hg_ref, ea_ref, ewr_ref, ebr_ref, w2_ref, b2_ref, s_ref,
             out_ref):
        hgv = hg_ref[...].astype(MXT)
        her = jnp.dot(ea_ref[...].astype(MXT), ewr_ref[...],
                      preferred_element_type=F32) + ebr_ref[...]
        if att_relu:
            her = jnp.maximum(her, 0.0)
        t = jnp.dot(hgv, w2_ref[...], preferred_element_type=F32)
        prod = her.astype(MXT) * t.astype(MXT)
        out_ref[...] = (
            jnp.dot(prod, s_ref[...], preferred_element_type=F32)
            + jnp.dot(hgv, b2_ref[...], preferred_element_type=F32))

    return pl.pallas_call(
        body,
        grid=(grid,),
        in_specs=[
            pl.BlockSpec((tile_rows, pack * in_c), lambda i: (i, 0)),
            pl.BlockSpec((tile_rows, attk), lambda i: (i, 0)),
            pl.BlockSpec((attk, wk), lambda i: (0, 0)),
            pl.BlockSpec((1, wk), lambda i: (0, 0)),
            pl.BlockSpec((pack * in_c, wk), lambda i: (0, 0)),
            pl.BlockSpec((pack * in_c, pack * HID), lambda i: (0, 0)),
            pl.BlockSpec((wk, pack * HID), lambda i: (0, 0)),
        ],
        out_specs=pl.BlockSpec((tile_rows, pack * HID), lambda i: (i, 0)),
        out_shape=jax.ShapeDtypeStruct((rows, pack * HID), F32),
    )(hg_p, att, attw.astype(MXT), attb, w2big, b2big,
      sbig).reshape(n_edges, HID)


def _recip(x):
    r = 1.0 / x
    return r * (2.0 - x * r)


def _rsqrt_nr(x):
    r = lax.rsqrt(x)
    return r * (1.5 - 0.5 * x * r * r)


def _tc_node(parts, cnts, h, root, bias, bn_g, bn_b, residual):
    n_nodes = h.shape[0]

    def body(p_ref, c_ref, h_ref, root_ref, bias_ref, g_ref, b_ref, out_ref):
        sums = p_ref[0] + p_ref[1]
        cnt = jnp.maximum(c_ref[0] + c_ref[1], 1.0)
        hv = h_ref[...]
        xn = (sums * _recip(cnt)
              + jnp.dot(hv, root_ref[...], preferred_element_type=F32)
              + bias_ref[...])
        mean = jnp.mean(xn, axis=0, keepdims=True)
        ctr = xn - mean
        var = jnp.mean(ctr * ctr, axis=0, keepdims=True)
        xn = ctr * _rsqrt_nr(var + 1e-5) * g_ref[...] + b_ref[...]
        xn = jnp.maximum(xn, 0.0)
        if residual:
            xn = xn + hv
        out_ref[...] = xn

    return pl.pallas_call(
        body,
        out_shape=jax.ShapeDtypeStruct((n_nodes, HID), F32),
    )(parts, cnts, h, root, bias.reshape(1, HID), bn_g.reshape(1, HID),
      bn_b.reshape(1, HID))


def _tc_pool(h, batch2, l1w, l1b, l2w, l2b, n_graphs):
    n_nodes = h.shape[0]

    def body(h_ref, b_ref, w1_ref, b1_ref, w2_ref, b2_ref, out_ref):
        gid = lax.broadcasted_iota(jnp.int32, (n_graphs, n_nodes), 0)
        oh = (gid == b_ref[...]).astype(F32)
        psum = jnp.dot(oh, h_ref[...], preferred_element_type=F32)
        cnt = jnp.maximum(jnp.sum(oh, axis=1, keepdims=True), 1.0)
        pooled = psum * _recip(cnt)
        h1 = jnp.maximum(
            jnp.dot(pooled, w1_ref[...], preferred_element_type=F32)
            + b1_ref[...], 0.0)
        out_ref[...] = (jnp.dot(h1, w2_ref[...], preferred_element_type=F32)
                        + b2_ref[...])

    return pl.pallas_call(
        body,
        out_shape=jax.ShapeDtypeStruct((n_graphs, 1), F32),
    )(h, batch2, l1w, l1b.reshape(1, -1), l2w, l2b.reshape(1, 1))


def kernel(x, edge_index, batch, edge_attr, params):
    n_nodes = x.shape[0]
    n_graphs = 64
    n_edges = edge_attr.shape[0]
    src1 = edge_index[0]
    dst2 = edge_index[1].reshape(-1, CHUNK)
    batch2 = batch.reshape(1, n_nodes)
    eat = edge_attr.T

    h = x
    cnts = None
    ea8 = None
    for l, p in enumerate(params["convs"]):
        in_c = h.shape[1]
        w2m = (p["ew2"].reshape(HID, in_c, HID)
               .transpose(1, 0, 2).reshape(in_c, HID * HID))
        b2m = p["eb2"].reshape(in_c, HID)
        if l == 0:
            hg, ea16 = _sc_gather(h, src1, eat)
            ea8 = ea16.reshape(n_edges // 8, 128)
        else:
            hg = _sc_gather(h, src1)
        if in_c == 64:
            he2 = _tc_he(ea8, p["ew1"], p["eb1"]).reshape(n_edges // 2, 32)
            attw = jnp.kron(jnp.eye(2, dtype=F32), _rep_mat())
            attb = jnp.zeros((1, 2 * HID * HID), F32)
            msg = _tc_edge(hg, he2, attw, attb, False, w2m, b2m, 2, 2000)
        else:
            ew1p = jnp.zeros((HID, HID), F32).at[:4].set(p["ew1"])
            attw = jnp.kron(jnp.eye(8, dtype=F32),
                            jnp.kron(ew1p, jnp.ones((1, HID), F32)))
            attb = jnp.tile(jnp.repeat(p["eb1"], HID), 8).reshape(1, -1)
            msg = _tc_edge(hg, ea8, attw, attb, True, w2m, b2m, 8, 1000)
        if l == 0:
            parts, cnts = _sc_scatter(msg, dst2, n_nodes, with_counts=True)
        else:
            (parts,) = _sc_scatter(msg, dst2, n_nodes, with_counts=False)
        h = _tc_node(parts, cnts, h, p["root"], p["bias"], p["bn_g"],
                     p["bn_b"], residual=(l > 0))
    return _tc_pool(h, batch2, params["lin1_w"], params["lin1_b"],
                    params["lin2_w"], params["lin2_b"], n_graphs)
